# Initial kernel scaffold; baseline (speedup 1.0000x reference)
import jax
import jax.numpy as jnp
from jax.experimental import pallas as pl


def kernel(queries, keys, values):
    raise NotImplementedError("write your pallas kernel here")


# speedup vs baseline: 1.9398x; 1.9398x over previous
import functools

import jax
import jax.numpy as jnp
import numpy as np
from jax.experimental import pallas as pl
from jax.experimental.pallas import tpu as pltpu

FACTOR = 5
NEG_INF = float("-inf")


def _head_kernel(c_ref, q_ref, k_ref, v_ref, o_ref,
                 m_ref, qr_ref, upd_ref, idx_ref, *, u, blk_q, blk_c):
    L, D = q_ref.shape
    scale = 1.0 / np.sqrt(D)

    n_blk = L // blk_q
    for b in range(n_blk):
        rows = pl.ds(b * blk_q, blk_q)
        s_blk = jax.lax.dot_general(
            q_ref[rows, :], k_ref[...],
            (((1,), (1,)), ((), ())),
            preferred_element_type=jnp.float32,
            precision=jax.lax.Precision.DEFAULT)
        cnt = c_ref[rows, :].astype(jnp.float32)
        mx = jnp.max(jnp.where(cnt > 0.0, s_blk, NEG_INF), axis=1)
        sm = jnp.sum(s_blk * cnt, axis=1)
        m_ref[:, rows] = (mx - sm * (1.0 / L))[None, :]

    lin = jax.lax.broadcasted_iota(jnp.int32, (1, L), 1)

    def topk_body(i, mv):
        mmax = jnp.max(mv)
        j = jnp.min(jnp.where(mv == mmax, lin, L))
        idx_ref[i] = j
        qr_ref[pl.ds(i, 1), :] = q_ref[pl.ds(j, 1), :]
        return jnp.where(lin == j, NEG_INF, mv)

    jax.lax.fori_loop(0, u, topk_body, m_ref[...], unroll=False)

    s2 = jax.lax.dot_general(
        qr_ref[...], k_ref[0:u, :],
        (((1,), (1,)), ((), ())),
        preferred_element_type=jnp.float32,
        precision=jax.lax.Precision.HIGHEST) * scale
    ri = jax.lax.broadcasted_iota(jnp.int32, (u, u), 0)
    ci = jax.lax.broadcasted_iota(jnp.int32, (u, u), 1)
    s2 = jnp.where(ci > ri, NEG_INF, s2)
    s2 = s2 - jnp.max(s2, axis=1, keepdims=True)
    e = jnp.exp(s2)
    attn = e / jnp.sum(e, axis=1, keepdims=True)
    upd_ref[...] = jax.lax.dot_general(
        attn, v_ref[0:u, :],
        (((1,), (0,)), ((), ())),
        preferred_element_type=jnp.float32,
        precision=jax.lax.Precision.HIGHEST)

    tri = (jax.lax.broadcasted_iota(jnp.int32, (blk_c, blk_c), 0)
           >= jax.lax.broadcasted_iota(jnp.int32, (blk_c, blk_c), 1)
           ).astype(jnp.float32)
    n_cblk = L // blk_c
    carry = jnp.zeros((1, D), jnp.float32)
    for b in range(n_cblk):
        rows = pl.ds(b * blk_c, blk_c)
        blk = jax.lax.dot_general(
            tri, v_ref[rows, :],
            (((1,), (0,)), ((), ())),
            preferred_element_type=jnp.float32,
            precision=jax.lax.Precision.HIGHEST) + carry
        o_ref[rows, :] = blk
        carry = blk[blk_c - 1:blk_c, :]

    def scat_body(i, _):
        j = idx_ref[i]
        o_ref[pl.ds(j, 1), :] = upd_ref[pl.ds(i, 1), :]
        return 0

    jax.lax.fori_loop(0, u, scat_body, 0, unroll=False)


def kernel(queries, keys, values):
    B, L, H, D = queries.shape
    L_K = keys.shape[1]
    U_part = min(int(FACTOR * np.ceil(np.log(L_K))), L_K)
    u = min(int(FACTOR * np.ceil(np.log(L))), L)
    assert U_part == u

    idx_sample = jax.random.randint(
        jax.random.key(42), (L, U_part), 0, L_K)
    counts = jnp.zeros((L, L_K), jnp.int8).at[
        jnp.arange(L)[:, None], idx_sample].add(1)

    qh = jnp.transpose(queries, (0, 2, 1, 3)).reshape(B * H, L, D)
    kh = jnp.transpose(keys, (0, 2, 1, 3)).reshape(B * H, L, D)
    vh = jnp.transpose(values, (0, 2, 1, 3)).reshape(B * H, L, D)

    blk_q, blk_c = 512, 256
    grid = (B * H,)
    out = pl.pallas_call(
        functools.partial(_head_kernel, u=u, blk_q=blk_q, blk_c=blk_c),
        grid=grid,
        in_specs=[
            pl.BlockSpec((L, L_K), lambda i: (0, 0)),
            pl.BlockSpec((None, L, D), lambda i: (i, 0, 0)),
            pl.BlockSpec((None, L, D), lambda i: (i, 0, 0)),
            pl.BlockSpec((None, L, D), lambda i: (i, 0, 0)),
        ],
        out_specs=pl.BlockSpec((None, L, D), lambda i: (i, 0, 0)),
        out_shape=jax.ShapeDtypeStruct((B * H, L, D), jnp.float32),
        scratch_shapes=[
            pltpu.VMEM((1, L), jnp.float32),
            pltpu.VMEM((u, D), jnp.float32),
            pltpu.VMEM((u, D), jnp.float32),
            pltpu.SMEM((u,), jnp.int32),
        ],
    )(counts, qh, kh, vh)

    return jnp.transpose(out.reshape(B, H, L, D), (0, 2, 1, 3))


# speedup vs baseline: 3.3580x; 1.7311x over previous
import functools

import jax
import jax.numpy as jnp
import numpy as np
from jax.experimental import pallas as pl
from jax.experimental.pallas import tpu as pltpu

FACTOR = 5
NEG_INF = float("-inf")


def _head_kernel(c_ref, q_ref, k_ref, v_ref, o_ref,
                 m_ref, qr_ref, upd_ref, idx_ref, *, u, blk_q, blk_c):
    L, D = q_ref.shape
    scale = 1.0 / np.sqrt(D)

    n_blk = L // blk_q
    for b in range(n_blk):
        rows = pl.ds(b * blk_q, blk_q)
        s_blk = jax.lax.dot_general(
            q_ref[rows, :], k_ref[...],
            (((1,), (1,)), ((), ())),
            preferred_element_type=jnp.float32,
            precision=jax.lax.Precision.DEFAULT)
        cnt = c_ref[rows, :].astype(jnp.float32)
        mx = jnp.max(jnp.where(cnt > 0.0, s_blk, NEG_INF), axis=1)
        sm = jnp.sum(s_blk * cnt, axis=1)
        m_ref[:, rows] = (mx - sm * (1.0 / L))[None, :]

    lin = jax.lax.broadcasted_iota(jnp.int32, (1, L), 1)

    def topk_body(i, mv):
        mmax = jnp.max(mv)
        j = jnp.min(jnp.where(mv == mmax, lin, L))
        idx_ref[i] = j
        qr_ref[pl.ds(i, 1), :] = q_ref[pl.ds(j, 1), :]
        return jnp.where(lin == j, NEG_INF, mv)

    jax.lax.fori_loop(0, u, topk_body, m_ref[...], unroll=False)

    s2 = jax.lax.dot_general(
        qr_ref[...], k_ref[0:u, :],
        (((1,), (1,)), ((), ())),
        preferred_element_type=jnp.float32,
        precision=jax.lax.Precision.HIGHEST) * scale
    ri = jax.lax.broadcasted_iota(jnp.int32, (u, u), 0)
    ci = jax.lax.broadcasted_iota(jnp.int32, (u, u), 1)
    s2 = jnp.where(ci > ri, NEG_INF, s2)
    s2 = s2 - jnp.max(s2, axis=1, keepdims=True)
    e = jnp.exp(s2)
    attn = e / jnp.sum(e, axis=1, keepdims=True)
    upd_ref[...] = jax.lax.dot_general(
        attn, v_ref[0:u, :],
        (((1,), (0,)), ((), ())),
        preferred_element_type=jnp.float32,
        precision=jax.lax.Precision.HIGHEST)

    tri = (jax.lax.broadcasted_iota(jnp.int32, (blk_c, blk_c), 0)
           >= jax.lax.broadcasted_iota(jnp.int32, (blk_c, blk_c), 1)
           ).astype(jnp.float32)
    n_cblk = L // blk_c
    carry = jnp.zeros((1, D), jnp.float32)
    for b in range(n_cblk):
        rows = pl.ds(b * blk_c, blk_c)
        blk = jax.lax.dot_general(
            tri, v_ref[rows, :],
            (((1,), (0,)), ((), ())),
            preferred_element_type=jnp.float32,
            precision=jax.lax.Precision.HIGHEST) + carry
        o_ref[rows, :] = blk
        carry = blk[blk_c - 1:blk_c, :]

    def scat_body(i, _):
        j = idx_ref[i]
        o_ref[pl.ds(j, 1), :] = upd_ref[pl.ds(i, 1), :]
        return 0

    jax.lax.fori_loop(0, u, scat_body, 0, unroll=False)


def kernel(queries, keys, values):
    B, L, H, D = queries.shape
    L_K = keys.shape[1]
    U_part = min(int(FACTOR * np.ceil(np.log(L_K))), L_K)
    u = min(int(FACTOR * np.ceil(np.log(L))), L)
    assert U_part == u

    idx_sample = jax.random.randint(
        jax.random.key(42), (L, U_part), 0, L_K)
    kiota = jnp.arange(L_K, dtype=jnp.int32)[None, :]
    counts = jnp.zeros((L, L_K), jnp.int8)
    for s in range(U_part):
        counts = counts + (idx_sample[:, s:s + 1] == kiota).astype(jnp.int8)

    qh = jnp.transpose(queries, (0, 2, 1, 3)).reshape(B * H, L, D)
    kh = jnp.transpose(keys, (0, 2, 1, 3)).reshape(B * H, L, D)
    vh = jnp.transpose(values, (0, 2, 1, 3)).reshape(B * H, L, D)

    blk_q, blk_c = 512, 256
    grid = (B * H,)
    out = pl.pallas_call(
        functools.partial(_head_kernel, u=u, blk_q=blk_q, blk_c=blk_c),
        grid=grid,
        in_specs=[
            pl.BlockSpec((L, L_K), lambda i: (0, 0)),
            pl.BlockSpec((None, L, D), lambda i: (i, 0, 0)),
            pl.BlockSpec((None, L, D), lambda i: (i, 0, 0)),
            pl.BlockSpec((None, L, D), lambda i: (i, 0, 0)),
        ],
        out_specs=pl.BlockSpec((None, L, D), lambda i: (i, 0, 0)),
        out_shape=jax.ShapeDtypeStruct((B * H, L, D), jnp.float32),
        scratch_shapes=[
            pltpu.VMEM((1, L), jnp.float32),
            pltpu.VMEM((u, D), jnp.float32),
            pltpu.VMEM((u, D), jnp.float32),
            pltpu.SMEM((u,), jnp.int32),
        ],
    )(counts, qh, kh, vh)

    return jnp.transpose(out.reshape(B, H, L, D), (0, 2, 1, 3))
